# Initial kernel scaffold; baseline (speedup 1.0000x reference)
import jax
import jax.numpy as jnp
from jax.experimental import pallas as pl


def kernel(x, adj_mat, W1, b1, W2, b2):
    raise NotImplementedError("write your pallas kernel here")


# speedup vs baseline: 12.0636x; 12.0636x over previous
import functools

import jax
import jax.numpy as jnp
from jax import lax
from jax.experimental import pallas as pl
from jax.experimental.pallas import tpu as pltpu
from jax.experimental.pallas import tpu_sc as plsc

N = 10000
E = 320000
D = 128
NC = 2
NS = 16
NW = NC * NS
EPW = E // NW
K = 80
CHUNKS = EPW // K
NA = 10240
RPT = NA // NS
ZR = 128

_mesh = plsc.VectorSubcoreMesh(core_axis_name="c", subcore_axis_name="s")


def _fill_zero(ref, rows, width):
    zero16 = jnp.zeros((16,), jnp.float32)

    def body(t, _):
        i = t // (width // 16)
        k = t % (width // 16)
        ref[i, pl.ds(k * 16, 16)] = zero16
        return 0

    lax.fori_loop(0, rows * (width // 16), body, 0)


def _fill_const(ref, rows, width, vec16):
    def body(t, _):
        i = t // (width // 16)
        k = t % (width // 16)
        ref[i, pl.ds(k * 16, 16)] = vec16
        return 0

    lax.fori_loop(0, rows * (width // 16), body, 0)


@functools.partial(
    pl.kernel,
    out_type=jax.ShapeDtypeStruct((NC, NA, D), jnp.float32),
    mesh=_mesh,
    scratch_types=[
        pltpu.VMEM((K,), jnp.int32),
        pltpu.VMEM((K, D), jnp.float32),
        pltpu.VMEM((ZR, D), jnp.float32),
        pltpu.VMEM_SHARED((NA, D), jnp.float32),
    ],
)
def _sc_degree(dst_hbm, out_hbm, idx_v, ones_v, zb_v, acc_sh):
    c = lax.axis_index("c")
    s = lax.axis_index("s")
    wid = c * NS + s

    _fill_const(ones_v, K, D, jnp.ones((16,), jnp.float32))
    _fill_zero(zb_v, ZR, D)

    def zcp(j, _):
        pltpu.sync_copy(zb_v, acc_sh.at[pl.ds(s * RPT + j * ZR, ZR)])
        return 0

    lax.fori_loop(0, RPT // ZR, zcp, 0)
    plsc.subcore_barrier()

    def body(j, _):
        e0 = wid * EPW + j * K
        pltpu.sync_copy(dst_hbm.at[pl.ds(e0, K)], idx_v)
        pltpu.sync_copy(ones_v, acc_sh.at[idx_v], add=True)
        return 0

    lax.fori_loop(0, CHUNKS, body, 0)
    plsc.subcore_barrier()

    pltpu.sync_copy(acc_sh.at[pl.ds(s * RPT, RPT)],
                    out_hbm.at[c, pl.ds(s * RPT, RPT)])


@functools.partial(
    pl.kernel,
    out_type=jax.ShapeDtypeStruct((NC, NA, D), jnp.float32),
    mesh=_mesh,
    scratch_types=[
        pltpu.VMEM((K,), jnp.int32),
        pltpu.VMEM((K,), jnp.int32),
        pltpu.VMEM((K, D), jnp.float32),
        pltpu.VMEM((ZR, D), jnp.float32),
        pltpu.VMEM_SHARED((NA, D), jnp.float32),
        pltpu.SemaphoreType.DMA,
    ],
)
def _sc_scatter(src_hbm, dst_hbm, h_hbm, out_hbm,
                idxs_v, idxd_v, rows_v, zb_v, acc_sh, sem):
    c = lax.axis_index("c")
    s = lax.axis_index("s")
    wid = c * NS + s

    _fill_zero(zb_v, ZR, D)

    def zcp(j, _):
        pltpu.sync_copy(zb_v, acc_sh.at[pl.ds(s * RPT + j * ZR, ZR)])
        return 0

    lax.fori_loop(0, RPT // ZR, zcp, 0)
    plsc.subcore_barrier()

    def body(j, _):
        e0 = wid * EPW + j * K
        pltpu.sync_copy(src_hbm.at[pl.ds(e0, K)], idxs_v)
        pltpu.sync_copy(dst_hbm.at[pl.ds(e0, K)], idxd_v)
        pltpu.async_copy(h_hbm.at[idxs_v], rows_v, sem).wait()
        pltpu.sync_copy(rows_v, acc_sh.at[idxd_v], add=True)
        return 0

    lax.fori_loop(0, CHUNKS, body, 0)
    plsc.subcore_barrier()

    pltpu.sync_copy(acc_sh.at[pl.ds(s * RPT, RPT)],
                    out_hbm.at[c, pl.ds(s * RPT, RPT)])


R = 2000


def _dinv_of(d0_ref, d1_ref):
    deg = d0_ref[:, 0:1] + d1_ref[:, 0:1] + 1.0
    return lax.rsqrt(deg)


def _tc_h1_body(x_ref, w_ref, d0_ref, d1_ref, o_ref):
    dinv = _dinv_of(d0_ref, d1_ref)
    o_ref[...] = jnp.dot(x_ref[...], w_ref[...],
                         preferred_element_type=jnp.float32) * dinv


def _tc_mid_body(a0_ref, a1_ref, hp_ref, d0_ref, d1_ref, b_ref, w_ref, o_ref):
    dinv = _dinv_of(d0_ref, d1_ref)
    out1 = (a0_ref[...] + a1_ref[...] + hp_ref[...]) * dinv + b_ref[...]
    u = jnp.maximum(out1, 0.0)
    o_ref[...] = jnp.dot(u, w_ref[...],
                         preferred_element_type=jnp.float32) * dinv


def _tc_final_body(a0_ref, a1_ref, hp_ref, d0_ref, d1_ref, b_ref, o_ref):
    dinv = _dinv_of(d0_ref, d1_ref)
    o_ref[...] = (a0_ref[...] + a1_ref[...] + hp_ref[...]) * dinv + b_ref[...]


_row_spec = pl.BlockSpec((R, D), lambda i: (i, 0))
_deg_spec = pl.BlockSpec((R, D), lambda i: (i, 0))
_w_spec = pl.BlockSpec((D, D), lambda i: (0, 0))
_b_spec = pl.BlockSpec((1, D), lambda i: (0, 0))
_out_sds = jax.ShapeDtypeStruct((N, D), jnp.float32)


def _tc_h1(x, W1, d0, d1):
    return pl.pallas_call(
        _tc_h1_body, grid=(N // R,),
        in_specs=[_row_spec, _w_spec, _deg_spec, _deg_spec],
        out_specs=_row_spec, out_shape=_out_sds)(x, W1, d0, d1)


def _tc_mid(a0, a1, hp, d0, d1, b, W2):
    return pl.pallas_call(
        _tc_mid_body, grid=(N // R,),
        in_specs=[_row_spec, _row_spec, _row_spec, _deg_spec, _deg_spec,
                  _b_spec, _w_spec],
        out_specs=_row_spec, out_shape=_out_sds)(a0, a1, hp, d0, d1, b, W2)


def _tc_final(a0, a1, hp, d0, d1, b):
    return pl.pallas_call(
        _tc_final_body, grid=(N // R,),
        in_specs=[_row_spec, _row_spec, _row_spec, _deg_spec, _deg_spec,
                  _b_spec],
        out_specs=_row_spec, out_shape=_out_sds)(a0, a1, hp, d0, d1, b)


def kernel(x, adj_mat, W1, b1, W2, b2):
    src = adj_mat[0]
    dst = adj_mat[1]
    degp = _sc_degree(dst)
    d0, d1 = degp[0, :N], degp[1, :N]
    h1p = _tc_h1(x, W1, d0, d1)
    acc1 = _sc_scatter(src, dst, h1p)
    h2p = _tc_mid(acc1[0, :N], acc1[1, :N], h1p, d0, d1,
                  b1.reshape(1, D), W2)
    acc2 = _sc_scatter(src, dst, h2p)
    return _tc_final(acc2[0, :N], acc2[1, :N], h2p, d0, d1, b2.reshape(1, D))
